# baseline (speedup 1.0000x reference)
import functools

import jax
import jax.numpy as jnp
from jax import lax
from jax.experimental import pallas as pl
from jax.experimental.pallas import tpu as pltpu
from jax.experimental.pallas import tpu_sc as plsc

VOCAB = 100000
GOVS = 50
EMBED = 16
BATCH = 1024
CTX = 40

_PACK = 128 // EMBED
_CHUNK = 128
_VT = 2048


_NCHK = 781
_PROWS = 12512


def _pack_sc(wt, tail):
    info = plsc.get_sparse_core_info()
    nc, ns = info.num_cores, info.num_subcores
    nw = nc * ns
    ntmax = -(-_NCHK // nw)
    nbig = _NCHK - nw * (ntmax - 1)
    mesh = plsc.VectorSubcoreMesh(core_axis_name="c", subcore_axis_name="s")

    @functools.partial(
        pl.kernel,
        out_type=jax.ShapeDtypeStruct((_PROWS, 128), jnp.float32),
        mesh=mesh,
        scratch_types=[
            pltpu.VMEM((4, 16, 128), jnp.float32),
            pltpu.VMEM(((ntmax) * 16, 128), jnp.float32),
            pltpu.VMEM((8, 128), jnp.float32),
            pltpu.SemaphoreType.DMA,
            pltpu.SemaphoreType.DMA,
            pltpu.SemaphoreType.DMA,
            pltpu.SemaphoreType.DMA,
        ],
        compiler_params=pltpu.CompilerParams(needs_layout_passes=False),
    )
    def pack(wt_hbm, tail_hbm, out_hbm, in_v, out_v, tail_v, s0, s1, s2, s3):
        wid = lax.axis_index("s") * nc + lax.axis_index("c")
        big = wid < nbig
        a = jnp.where(big, ntmax * wid,
                      ntmax * nbig + (ntmax - 1) * (wid - nbig))
        nt = jnp.where(big, ntmax, ntmax - 1)
        sems = (s0, s1, s2, s3)
        lanes0 = lax.broadcasted_iota(jnp.int32, (16,), 0)

        def dma(t, start):
            k = a + t
            slot = t % 4
            full = pltpu.make_async_copy(
                wt_hbm.at[:, pl.ds(128 * k, 128)], in_v.at[slot], sems[slot])

            @pl.when(t < nt)
            def _():
                full.start() if start else full.wait()

        for t in range(3):
            dma(t, True)
        for t in range(ntmax):
            if t + 3 < ntmax:
                dma(t + 3, True)
            dma(t, False)

            @pl.when(t < nt)
            def _():
                def jbody(j, _):
                    for p in range(8):
                        v = plsc.load_gather(
                            in_v, [jnp.full((16,), t % 4, jnp.int32),
                                   lanes0,
                                   jnp.zeros((16,), jnp.int32) + (j * 8 + p)])
                        out_v[t * 16 + j, pl.ds(p * 16, 16)] = v
                    return 0
                lax.fori_loop(0, 16, jbody, 0)

        @pl.when(big)
        def _():
            pltpu.sync_copy(out_v, out_hbm.at[pl.ds(16 * a, ntmax * 16)])

        @pl.when(jnp.logical_not(big))
        def _():
            pltpu.sync_copy(out_v.at[pl.ds(0, (ntmax - 1) * 16)],
                            out_hbm.at[pl.ds(16 * a, (ntmax - 1) * 16)])

        @pl.when(wid == nw - 1)
        def _():
            pltpu.sync_copy(tail_hbm, tail_v)
            pltpu.sync_copy(tail_v, out_hbm.at[pl.ds(_NCHK * 16, 8)])

    return pack(wt, tail)


def _combine_sc(context, gov, wp, gp):
    info = plsc.get_sparse_core_info()
    nc, ns = info.num_cores, info.num_subcores
    nw = nc * ns
    bpw = BATCH // nw
    ipw = bpw * CTX
    nch = ipw // _CHUNK
    ctx_flat = context.reshape(BATCH * CTX)
    ngp = gp.shape[0]

    mesh = plsc.VectorSubcoreMesh(core_axis_name="c", subcore_axis_name="s")

    @functools.partial(
        pl.kernel,
        out_type=jax.ShapeDtypeStruct((BATCH, EMBED), jnp.float32),
        mesh=mesh,
        scratch_types=[
            pltpu.VMEM((ipw,), jnp.int32),
            pltpu.VMEM((ipw,), jnp.int32),
            pltpu.VMEM((2, _CHUNK, 128), jnp.float32),
            pltpu.VMEM((bpw,), jnp.int32),
            pltpu.VMEM((ngp, 128), jnp.float32),
            pltpu.VMEM((bpw, EMBED), jnp.float32),
            pltpu.SemaphoreType.DMA,
            pltpu.SemaphoreType.DMA,
        ],
        compiler_params=pltpu.CompilerParams(needs_layout_passes=False),
    )
    def combine(ctx_hbm, gov_hbm, wp_hbm, gp_hbm, out_hbm,
                idx_v, row_v, rows_v, gidx_v, gtbl_v, out_v, sem0, sem1):
        wid = lax.axis_index("s") * nc + lax.axis_index("c")
        pltpu.sync_copy(ctx_hbm.at[pl.ds(wid * ipw, ipw)], idx_v)
        pltpu.sync_copy(gov_hbm.at[pl.ds(wid * bpw, bpw)], gidx_v)
        pltpu.sync_copy(gp_hbm, gtbl_v)

        def rowid_body(k, _):
            row_v[pl.ds(k * 16, 16)] = lax.shift_right_logical(
                idx_v[pl.ds(k * 16, 16)], 3)
            return 0
        lax.fori_loop(0, ipw // 16, rowid_body, 0)

        def zero_body(r, _):
            out_v[r, :] = jnp.zeros((EMBED,), jnp.float32)
            return 0
        lax.fori_loop(0, bpw, zero_body, 0)

        sems = (sem0, sem1)

        def fire(c):
            return pltpu.async_copy(
                wp_hbm.at[row_v.at[pl.ds(c * _CHUNK, _CHUNK)]],
                rows_v.at[c % 2], sems[c % 2])

        lanes0 = lax.broadcasted_iota(jnp.int32, (16,), 0)
        zeros16 = jnp.zeros((16,), jnp.int32)
        pending = fire(0)
        for c in range(nch):
            nxt = fire(c + 1) if c + 1 < nch else None
            pending.wait()
            pending = nxt
            buf = jnp.full((16,), c % 2, jnp.int32)

            def grp_body(q, _):
                base = q * 16
                ivec = idx_v[pl.ds(c * _CHUNK + base, 16)]
                for t in range(16):
                    s = ivec[t]
                    val = plsc.load_gather(
                        rows_v, [buf, zeros16 + (base + t),
                                 lanes0 + (s & (_PACK - 1)) * EMBED])
                    r = (c * _CHUNK + base + t) // CTX
                    out_v[r, :] = out_v[r, :] + val
                return 0
            lax.fori_loop(0, _CHUNK // 16, grp_body, 0)

        def final_body(q, _):
            gvec = gidx_v[pl.ds(q * 16, 16)]
            for t in range(16):
                g = gvec[t]
                r = q * 16 + t
                gval = plsc.load_gather(
                    gtbl_v, [zeros16 + lax.shift_right_logical(g, 3),
                             lanes0 + (g & (_PACK - 1)) * EMBED])
                out_v[r, :] = out_v[r, :] * (1.0 / CTX) + gval
            return 0
        lax.fori_loop(0, bpw // 16, final_body, 0)

        pltpu.sync_copy(out_v, out_hbm.at[pl.ds(wid * bpw, bpw)])

    return combine(ctx_flat, gov, wp, gp)


def _project_tc_t(comb_aug, w_aug_t):
    nvt = pl.cdiv(VOCAB, _VT)
    ka = comb_aug.shape[1]

    def mm(w_ref, comb_ref, out_ref):
        out_ref[...] = lax.dot_general(
            w_ref[...], comb_ref[...],
            dimension_numbers=(((0,), (1,)), ((), ())),
            preferred_element_type=jnp.float32,
        )

    return pl.pallas_call(
        mm,
        grid=(nvt,),
        in_specs=[
            pl.BlockSpec((ka, _VT), lambda i: (0, i)),
            pl.BlockSpec((BATCH, ka), lambda i: (0, 0)),
        ],
        out_specs=pl.BlockSpec((_VT, BATCH), lambda i: (i, 0)),
        out_shape=jax.ShapeDtypeStruct((VOCAB, BATCH), jnp.float32),
    )(w_aug_t, comb_aug)


def kernel(context, gov, word_emb, gov_emb, W, b):
    tail = jnp.pad(word_emb[_NCHK * 128:], ((0, 32), (0, 0))).reshape(8, 128)
    wp = _pack_sc(word_emb.T, tail)
    gpad = -GOVS % (_PACK * 8)
    gp = jnp.pad(gov_emb, ((0, gpad), (0, 0))).reshape(-1, 128)
    combined = _combine_sc(context, gov, wp, gp)
    comb_aug = jnp.concatenate(
        [combined, jnp.ones((BATCH, 1), jnp.float32)], axis=1)
    w_aug_t = jnp.concatenate([W.T, b[None, :]], axis=0)
    return _project_tc_t(comb_aug, w_aug_t).T


# speedup vs baseline: 1.2879x; 1.2879x over previous
import functools

import jax
import jax.numpy as jnp
from jax import lax
from jax.experimental import pallas as pl
from jax.experimental.pallas import tpu as pltpu
from jax.experimental.pallas import tpu_sc as plsc

VOCAB = 100000
GOVS = 50
EMBED = 16
BATCH = 1024
CTX = 40

_IDX_CHUNK = 128
_VT = 2048
_NCHK = 781


def _pack_sc(wt, tail):
    info = plsc.get_sparse_core_info()
    nc, ns = info.num_cores, info.num_subcores
    nw = nc * ns
    ntmax = -(-_NCHK // nw)
    nbig = _NCHK - nw * (ntmax - 1)
    mesh = plsc.VectorSubcoreMesh(core_axis_name="c", subcore_axis_name="s")

    @functools.partial(
        pl.kernel,
        out_type=jax.ShapeDtypeStruct((VOCAB * EMBED,), jnp.float32),
        mesh=mesh,
        scratch_types=[
            pltpu.VMEM((4, 16, 128), jnp.float32),
            pltpu.VMEM((ntmax * 16 * 128,), jnp.float32),
            pltpu.VMEM((32 * EMBED,), jnp.float32),
            pltpu.SemaphoreType.DMA,
            pltpu.SemaphoreType.DMA,
            pltpu.SemaphoreType.DMA,
            pltpu.SemaphoreType.DMA,
        ],
        compiler_params=pltpu.CompilerParams(needs_layout_passes=False),
    )
    def pack(wt_hbm, tail_hbm, out_hbm, in_v, out_v, tail_v, s0, s1, s2, s3):
        wid = lax.axis_index("s") * nc + lax.axis_index("c")
        big = wid < nbig
        a = jnp.where(big, ntmax * wid,
                      ntmax * nbig + (ntmax - 1) * (wid - nbig))
        nt = jnp.where(big, ntmax, ntmax - 1)
        sems = (s0, s1, s2, s3)
        lanes0 = lax.broadcasted_iota(jnp.int32, (16,), 0)

        def dma(t, start):
            k = a + t
            slot = t % 4
            cp = pltpu.make_async_copy(
                wt_hbm.at[:, pl.ds(128 * k, 128)], in_v.at[slot], sems[slot])

            @pl.when(t < nt)
            def _():
                cp.start() if start else cp.wait()

        for t in range(3):
            dma(t, True)
        for t in range(ntmax):
            if t + 3 < ntmax:
                dma(t + 3, True)
            dma(t, False)

            @pl.when(t < nt)
            def _():
                @functools.partial(plsc.parallel_loop, 0, 16)
                def _jloop(j):
                    for p in range(8):
                        v = plsc.load_gather(
                            in_v, [jnp.full((16,), t % 4, jnp.int32),
                                   lanes0,
                                   jnp.zeros((16,), jnp.int32) + (j * 8 + p)])
                        out_v[pl.ds(t * 2048 + j * 128 + p * 16, 16)] = v

        @pl.when(big)
        def _():
            pltpu.sync_copy(out_v, out_hbm.at[pl.ds(2048 * a, ntmax * 2048)])

        @pl.when(jnp.logical_not(big))
        def _():
            pltpu.sync_copy(out_v.at[pl.ds(0, (ntmax - 1) * 2048)],
                            out_hbm.at[pl.ds(2048 * a, (ntmax - 1) * 2048)])

        @pl.when(wid == nw - 1)
        def _():
            pltpu.sync_copy(tail_hbm, tail_v)
            pltpu.sync_copy(tail_v, out_hbm.at[pl.ds(_NCHK * 2048, 512)])

    return pack(wt, tail)


def _combine_sc(context, gov, wl, gov_emb):
    info = plsc.get_sparse_core_info()
    nc, ns = info.num_cores, info.num_subcores
    nw = nc * ns
    bpw = BATCH // nw
    ipw = bpw * CTX
    nch = ipw // _IDX_CHUNK
    ctx_flat = context.reshape(BATCH * CTX)

    mesh = plsc.VectorSubcoreMesh(core_axis_name="c", subcore_axis_name="s")

    @functools.partial(
        pl.kernel,
        out_type=jax.ShapeDtypeStruct((BATCH, EMBED), jnp.float32),
        mesh=mesh,
        scratch_types=[
            pltpu.VMEM((ipw,), jnp.int32),
            pltpu.VMEM((ipw, EMBED), jnp.float32),
            pltpu.VMEM((bpw,), jnp.int32),
            pltpu.VMEM((bpw, EMBED), jnp.float32),
            pltpu.VMEM((bpw, EMBED), jnp.float32),
            pltpu.SemaphoreType.DMA,
        ],
        compiler_params=pltpu.CompilerParams(use_tc_tiling_on_sc=False),
    )
    def combine(ctx_hbm, gov_hbm, wl_hbm, gemb_hbm, out_hbm,
                idx_v, rows_v, gidx_v, grows_v, out_v, sem):
        wid = lax.axis_index("s") * nc + lax.axis_index("c")
        pltpu.sync_copy(ctx_hbm.at[pl.ds(wid * ipw, ipw)], idx_v)
        pltpu.sync_copy(gov_hbm.at[pl.ds(wid * bpw, bpw)], gidx_v)
        copies = [
            pltpu.async_copy(wl_hbm.at[idx_v.at[pl.ds(k * _IDX_CHUNK,
                                                      _IDX_CHUNK)]],
                             rows_v.at[pl.ds(k * _IDX_CHUNK, _IDX_CHUNK)],
                             sem)
            for k in range(nch)
        ]
        copies.append(pltpu.async_copy(gemb_hbm.at[gidx_v], grows_v, sem))
        for c in copies:
            c.wait()

        def row_body(r, _):
            def acc_body(j, acc):
                return acc + rows_v[r * CTX + j, :]
            s = lax.fori_loop(0, CTX, acc_body,
                              jnp.zeros((EMBED,), jnp.float32))
            out_v[r, :] = s * (1.0 / CTX) + grows_v[r, :]
            return 0

        lax.fori_loop(0, bpw, row_body, 0)
        pltpu.sync_copy(out_v, out_hbm.at[pl.ds(wid * bpw, bpw)])

    return combine(ctx_flat, gov, wl, gov_emb)


def _project_tc_t(comb_aug, w_aug_t):
    nvt = pl.cdiv(VOCAB, _VT)
    ka = comb_aug.shape[1]

    def mm(w_ref, comb_ref, out_ref):
        out_ref[...] = lax.dot_general(
            w_ref[...], comb_ref[...],
            dimension_numbers=(((0,), (1,)), ((), ())),
            preferred_element_type=jnp.float32,
        )

    return pl.pallas_call(
        mm,
        grid=(nvt,),
        in_specs=[
            pl.BlockSpec((ka, _VT), lambda i: (0, i)),
            pl.BlockSpec((BATCH, ka), lambda i: (0, 0)),
        ],
        out_specs=pl.BlockSpec((_VT, BATCH), lambda i: (i, 0)),
        out_shape=jax.ShapeDtypeStruct((VOCAB, BATCH), jnp.float32),
    )(w_aug_t, comb_aug)


def kernel(context, gov, word_emb, gov_emb, W, b):
    tail = word_emb[_NCHK * 128:].reshape(32 * EMBED)
    wl = _pack_sc(word_emb.T, tail).reshape(VOCAB, EMBED)
    combined = _combine_sc(context, gov, wl, gov_emb)
    comb_aug = jnp.concatenate(
        [combined, jnp.ones((BATCH, 1), jnp.float32)], axis=1)
    w_aug_t = jnp.concatenate([W.T, b[None, :]], axis=0)
    return _project_tc_t(comb_aug, w_aug_t).T
